# baseline (speedup 1.0000x reference)
import functools

import jax
import jax.numpy as jnp
from jax import lax
from jax.experimental import pallas as pl
from jax.experimental.pallas import tpu as pltpu
from jax.experimental.pallas import tpu_sc as plsc

N_FIELDS = 26
VOCAB = 100000
N_DIM = 32
BATCH = 4096
L = 200

NC = 2
NS = 16
NW = NC * NS
B_PER_W = BATCH // NW
CHUNK = 128
N_CHUNKS = B_PER_W * N_FIELDS // CHUNK


def _body(tab_hbm, idx_hbm, feat_hbm, out_hbm, idx_v, emb_v, gsem, fsem):
    wid = lax.axis_index("s") * NC + lax.axis_index("c")
    b0 = wid * B_PER_W
    pltpu.sync_copy(idx_hbm.at[wid], idx_v)
    fh = []
    for k in range(B_PER_W):
        fh.append(
            pltpu.async_copy(
                feat_hbm.at[b0 + k],
                out_hbm.at[b0 + k, pl.ds(N_FIELDS, L)],
                fsem,
            )
        )
    gh = []
    for j in range(N_CHUNKS):
        gh.append(
            pltpu.async_copy(
                tab_hbm.at[idx_v.at[j]],
                emb_v.at[pl.ds(j * CHUNK, CHUNK)],
                gsem,
            )
        )
    for h in gh:
        h.wait()
    for k in range(B_PER_W):
        pltpu.sync_copy(
            emb_v.at[pl.ds(k * N_FIELDS, N_FIELDS)],
            out_hbm.at[b0 + k, pl.ds(0, N_FIELDS)],
        )
    for h in fh:
        h.wait()


_fused = functools.partial(
    pl.kernel,
    mesh=plsc.VectorSubcoreMesh(core_axis_name="c", subcore_axis_name="s"),
    compiler_params=pltpu.CompilerParams(use_tc_tiling_on_sc=False),
    out_type=jax.ShapeDtypeStruct((BATCH, N_FIELDS + L, N_DIM), jnp.float32),
    scratch_types=[
        pltpu.VMEM((N_CHUNKS, CHUNK), jnp.int32),
        pltpu.VMEM((B_PER_W * N_FIELDS, N_DIM), jnp.float32),
        pltpu.SemaphoreType.DMA,
        pltpu.SemaphoreType.DMA,
    ],
)(_body)


def kernel(feature, indices, tables):
    tab_flat = tables.reshape(N_FIELDS * VOCAB, N_DIM)
    flat_idx = indices.astype(jnp.int32) + (
        jnp.arange(N_FIELDS, dtype=jnp.int32) * VOCAB
    )[None, :]
    flat_idx = flat_idx.reshape(NW, N_CHUNKS, CHUNK)
    return _fused(tab_flat, flat_idx, feature)


# speedup vs baseline: 2.6001x; 2.6001x over previous
import functools

import jax
import jax.numpy as jnp
from jax import lax
from jax.experimental import pallas as pl
from jax.experimental.pallas import tpu as pltpu
from jax.experimental.pallas import tpu_sc as plsc

N_FIELDS = 26
VOCAB = 100000
N_DIM = 32
BATCH = 4096
L = 200

NC = 2
NS = 16
NW = NC * NS
B_PER_W = BATCH // NW
KB = 4
NF_IT = B_PER_W // KB
RING = 4
AHEAD = 2


def _body(tab3, idxh, feat3, out3, idx_v, gbuf, fbuf, *sems):
    gsem = sems[0:RING]
    esem = sems[RING:2 * RING]
    lsem = sems[2 * RING:3 * RING]
    ssem = sems[3 * RING:4 * RING]
    wid = lax.axis_index("s") * NC + lax.axis_index("c")
    b0 = wid * B_PER_W
    pltpu.sync_copy(idxh.at[wid], idx_v)

    def gfire(f):
        return pltpu.async_copy(
            tab3.at[f].at[idx_v.at[f]], gbuf.at[f % RING], gsem[f % RING])

    def efire(f):
        return pltpu.async_copy(
            gbuf.at[f % RING], out3.at[pl.ds(b0, B_PER_W), f], esem[f % RING])

    def lfire(g):
        return pltpu.async_copy(
            feat3.at[pl.ds(b0 + g * KB, KB)], fbuf.at[g % RING],
            lsem[g % RING])

    def sfire(g):
        return pltpu.async_copy(
            fbuf.at[g % RING],
            out3.at[pl.ds(b0 + g * KB, KB), pl.ds(N_FIELDS, L)],
            ssem[g % RING])

    gpend = [gfire(f) for f in range(AHEAD)]
    lpend = [lfire(g) for g in range(AHEAD)]
    epend = [None] * RING
    spend = [None] * RING

    for f in range(N_FIELDS):
        nf = f + AHEAD
        if nf < N_FIELDS:
            if epend[nf % RING] is not None:
                epend[nf % RING].wait()
                epend[nf % RING] = None
            gpend.append(gfire(nf))
        gpend[0].wait()
        gpend = gpend[1:]
        epend[f % RING] = efire(f)

    for g in range(NF_IT):
        ng = g + AHEAD
        if ng < NF_IT:
            if spend[ng % RING] is not None:
                spend[ng % RING].wait()
                spend[ng % RING] = None
            lpend.append(lfire(ng))
        lpend[0].wait()
        lpend = lpend[1:]
        spend[g % RING] = sfire(g)

    for h in epend:
        if h is not None:
            h.wait()
    for h in spend:
        if h is not None:
            h.wait()


_fused = functools.partial(
    pl.kernel,
    mesh=plsc.VectorSubcoreMesh(core_axis_name="c", subcore_axis_name="s"),
    compiler_params=pltpu.CompilerParams(use_tc_tiling_on_sc=False),
    out_type=jax.ShapeDtypeStruct((BATCH, N_FIELDS + L, N_DIM), jnp.float32),
    scratch_types=[
        pltpu.VMEM((N_FIELDS, B_PER_W), jnp.int32),
        pltpu.VMEM((RING, B_PER_W, N_DIM), jnp.float32),
        pltpu.VMEM((RING, KB, L, N_DIM), jnp.float32),
    ] + [pltpu.SemaphoreType.DMA] * (4 * RING),
)(_body)


def kernel(feature, indices, tables):
    fi = indices.astype(jnp.int32).T.reshape(N_FIELDS, NW, B_PER_W)
    fi = fi.transpose(1, 0, 2)
    return _fused(tables, fi, feature)


# speedup vs baseline: 3.6477x; 1.4029x over previous
import functools

import jax
import jax.numpy as jnp
from jax import lax
from jax.experimental import pallas as pl
from jax.experimental.pallas import tpu as pltpu
from jax.experimental.pallas import tpu_sc as plsc

N_FIELDS = 26
VOCAB = 100000
N_DIM = 32
BATCH = 4096
L = 200

NC = 2
NS = 16
NW = NC * NS
B_PER_W = BATCH // NW
N_LINES = N_FIELDS * VOCAB // 4
LF = 8
NF_IT = L // LF


def _extract_plane(lines, offs, ttbuf):
    def do_c(c, _):
        for g in range(8):
            row = jnp.arange(16, dtype=jnp.int32) + (g * 16)
            col = offs.at[pl.ds(g * 16, 16)][...] + c
            v = plsc.load_gather(lines, [row, col])
            ttbuf.at[c, pl.ds(g * 16, 16)][...] = v
        return 0

    lax.fori_loop(0, N_DIM, do_c, 0, unroll=False)


def _body(tab_hbm, idx_hbm, off_hbm, feat_hbm, out_hbm,
          idx_v, off_v, l0, l1, tt0, tt1, fb0, fb1,
          g0sem, g1sem, t0sem, t1sem, f0sem, f1sem):
    wid = lax.axis_index("s") * NC + lax.axis_index("c")
    b0 = wid * B_PER_W
    pltpu.sync_copy(idx_hbm.at[wid], idx_v)
    pltpu.sync_copy(off_hbm.at[wid], off_v)

    lbufs = (l0, l1)
    tbufs = (tt0, tt1)
    gsems = (g0sem, g1sem)
    tsems = (t0sem, t1sem)
    fbufs = (fb0, fb1)
    fsems = (f0sem, f1sem)

    def gfire(f):
        return pltpu.async_copy(
            tab_hbm.at[idx_v.at[f]], lbufs[f % 2], gsems[f % 2])

    def tfire(f):
        return pltpu.async_copy(
            tbufs[f % 2], out_hbm.at[f, pl.ds(0, N_DIM), pl.ds(b0, B_PER_W)],
            tsems[f % 2])

    def ffire(g):
        return pltpu.async_copy(
            feat_hbm.at[pl.ds(g * LF, LF), pl.ds(0, N_DIM),
                        pl.ds(b0, B_PER_W)],
            fbufs[g % 2], fsems[g % 2])

    gpend = gfire(0)
    fpend = ffire(0)
    fstore = [None, None]
    tpend = [None, None]

    for f in range(N_FIELDS):
        if f + 1 < N_FIELDS:
            gnext = gfire(f + 1)
        gpend.wait()
        if tpend[f % 2] is not None:
            tpend[f % 2].wait()
            tpend[f % 2] = None
        _extract_plane(lbufs[f % 2], off_v.at[f], tbufs[f % 2])
        tpend[f % 2] = tfire(f)
        if f + 1 < N_FIELDS:
            gpend = gnext

    for g in range(NF_IT):
        if g + 1 < NF_IT:
            fnext = ffire(g + 1)
        fpend.wait()
        if fstore[g % 2] is not None:
            fstore[g % 2].wait()
            fstore[g % 2] = None
        fstore[g % 2] = pltpu.async_copy(
            fbufs[g % 2],
            out_hbm.at[pl.ds(N_FIELDS + g * LF, LF), pl.ds(0, N_DIM),
                       pl.ds(b0, B_PER_W)],
            fsems[g % 2])
        if g + 1 < NF_IT:
            fpend = fnext

    for h in tpend + fstore:
        if h is not None:
            h.wait()


_fused = functools.partial(
    pl.kernel,
    mesh=plsc.VectorSubcoreMesh(core_axis_name="c", subcore_axis_name="s"),
    compiler_params=pltpu.CompilerParams(
        use_tc_tiling_on_sc=True, needs_layout_passes=False),
    out_type=jax.ShapeDtypeStruct((N_FIELDS + L, N_DIM, BATCH), jnp.float32),
    scratch_types=[
        pltpu.VMEM((N_FIELDS, B_PER_W), jnp.int32),
        pltpu.VMEM((N_FIELDS, B_PER_W), jnp.int32),
        pltpu.VMEM((B_PER_W, 128), jnp.float32),
        pltpu.VMEM((B_PER_W, 128), jnp.float32),
        pltpu.VMEM((N_DIM, B_PER_W), jnp.float32),
        pltpu.VMEM((N_DIM, B_PER_W), jnp.float32),
        pltpu.VMEM((LF, N_DIM, B_PER_W), jnp.float32),
        pltpu.VMEM((LF, N_DIM, B_PER_W), jnp.float32),
    ] + [pltpu.SemaphoreType.DMA] * 6,
)(_body)


def kernel(feature, indices, tables):
    tab_lines = tables.reshape(N_LINES, 128)
    flat = indices.astype(jnp.int32) + (
        jnp.arange(N_FIELDS, dtype=jnp.int32) * VOCAB)[None, :]
    flat = flat.T.reshape(N_FIELDS, NW, B_PER_W).transpose(1, 0, 2)
    lines = flat // 4
    offs = (flat % 4) * N_DIM
    feature_t = jnp.transpose(feature, (1, 2, 0))
    out_t = _fused(tab_lines, lines, offs, feature_t)
    return jnp.transpose(out_t, (2, 0, 1))
